# baseline (speedup 1.0000x reference)
import functools

import jax
import jax.numpy as jnp
from jax import lax
from jax.experimental import pallas as pl
from jax.experimental.pallas import tpu as pltpu
from jax.experimental.pallas import tpu_sc as plsc

_N = 10000
_E = 320000
_D = 128
_H = 64
_NSUB = 16
_NCORE = 2
_EPT = _E // _NSUB
_B = 40
_NB = _EPT // _B
_G = 10
_NG = _NB // _G

_BN_NODE = 400
_NBN = _N // _BN_NODE
_BE = 512
_NBE = _E // _BE


def _node_mm_body(h_ref, aw, ab, bw, bb, dw, db, ew, eb, ah_ref, db2_ref, eh2_ref):
    hblk = h_ref[...]
    ah_ref[0] = jnp.dot(hblk, aw[0], preferred_element_type=jnp.float32) + ab[0]
    dh = jnp.dot(hblk, dw[0], preferred_element_type=jnp.float32) + db[0]
    bh = jnp.dot(hblk, bw[0], preferred_element_type=jnp.float32) + bb[0]
    db2_ref[:, :_H] = dh
    db2_ref[:, _H:] = bh
    eh2_ref[...] = jnp.dot(hblk, ew[...], preferred_element_type=jnp.float32) + eb[...]


def _node_mm(h, A_w, A_b2, B_w, B_b2, D_w, D_b2, E_w, E_b2):
    wspec = pl.BlockSpec((1, _D, _H), lambda nb, c: (c, 0, 0))
    bspec = pl.BlockSpec((1, 1, _H), lambda nb, c: (c, 0, 0))
    return pl.pallas_call(
        _node_mm_body,
        grid=(_NBN, _NCORE),
        in_specs=[
            pl.BlockSpec((_BN_NODE, _D), lambda nb, c: (nb, 0)),
            wspec, bspec, wspec, bspec, wspec, bspec,
            pl.BlockSpec((_D, _D), lambda nb, c: (0, 0)),
            pl.BlockSpec((1, _D), lambda nb, c: (0, 0)),
        ],
        out_specs=[
            pl.BlockSpec((1, _BN_NODE, _H), lambda nb, c: (c, nb, 0)),
            pl.BlockSpec((_BN_NODE, _D), lambda nb, c: (c * _NBN + nb, 0)),
            pl.BlockSpec((_BN_NODE, _D), lambda nb, c: (nb, 0)),
        ],
        out_shape=[
            jax.ShapeDtypeStruct((_NCORE, _N, _H), jnp.float32),
            jax.ShapeDtypeStruct((2 * _N, _D), jnp.float32),
            jax.ShapeDtypeStruct((_N, _D), jnp.float32),
        ],
    )(h, A_w, A_b2, B_w, B_b2, D_w, D_b2, E_w, E_b2)


def _edge_mm_body(e_ref, cw, cb, ce_ref):
    ce_ref[0] = jnp.dot(e_ref[...], cw[0], preferred_element_type=jnp.float32) + cb[0]


def _edge_mm(e, C_w, C_b2):
    return pl.pallas_call(
        _edge_mm_body,
        grid=(_NBE, _NCORE),
        in_specs=[
            pl.BlockSpec((_BE, _D), lambda ebk, c: (ebk, 0)),
            pl.BlockSpec((1, _D, _H), lambda ebk, c: (c, 0, 0)),
            pl.BlockSpec((1, 1, _H), lambda ebk, c: (c, 0, 0)),
        ],
        out_specs=pl.BlockSpec((1, _BE, _H), lambda ebk, c: (c, ebk, 0)),
        out_shape=jax.ShapeDtypeStruct((_NCORE, _E, _H), jnp.float32),
    )(e, C_w, C_b2)


_PHN = 5120
_PACC = 5248
_ZPT = _PACC // _NSUB
_RPT = _PHN // _NSUB


def _sc_body(db2, eh2, ce2, srcg, dstf, dsta, dstb,
             enew_out, val_out, acc_out,
             srcg_v, dstf_v, dstp_v, db_v, eh_v, ce_v, enew_v, val_v,
             acc_sh, sem1, sem2):
    c = lax.axis_index("c")
    s = lax.axis_index("s")
    zero16 = jnp.zeros((16,), jnp.float32)
    co = c * _H

    def _zero_acc():
        def _zrow(r, carry):
            for i in range(_D // 16):
                val_v[r, pl.ds(i * 16, 16)] = zero16
            return carry
        lax.fori_loop(0, _B, _zrow, 0)
        base = s * _ZPT
        for q in range(_ZPT // _B):
            pltpu.sync_copy(val_v, acc_sh.at[pl.ds(base + q * _B, _B)])
        rem = _ZPT - (_ZPT // _B) * _B
        pltpu.sync_copy(val_v.at[pl.ds(0, rem)],
                        acc_sh.at[pl.ds(base + (_ZPT // _B) * _B, rem)])

    def _drain(p):
        pltpu.sync_copy(acc_sh.at[pl.ds(s * _RPT, _RPT)],
                        acc_out.at[c, p, pl.ds(s * _RPT, _RPT)])

    _zero_acc()
    plsc.subcore_barrier()

    def _group1(g, carry):
        pltpu.sync_copy(srcg.at[c, s, g], srcg_v)
        pltpu.sync_copy(dstf.at[s, g], dstf_v)
        pltpu.sync_copy(dsta.at[s, g], dstp_v)

        def _batch1(j, carry2):
            row0 = s * _EPT + (g * _G + j) * _B
            pltpu.async_copy(db2.at[srcg_v.at[j]], db_v, sem1).wait()
            pltpu.async_copy(eh2.at[dstf_v.at[j]], eh_v, sem2).wait()
            pltpu.sync_copy(ce2.at[c, pl.ds(row0, _B)], ce_v)

            def _row(r, carry3):
                for i in range(_H // 16):
                    o = i * 16
                    d_ = db_v[r, pl.ds(o, 16)]
                    b_ = db_v[r, pl.ds(_H + o, 16)]
                    x = d_ + eh_v[r, pl.ds(co + o, 16)] + ce_v[r, pl.ds(o, 16)]
                    enew_v[r, pl.ds(o, 16)] = x
                    sg = 1.0 / (1.0 + jnp.exp(-x))
                    val_v[r, pl.ds(o, 16)] = sg * b_
                    val_v[r, pl.ds(_H + o, 16)] = sg
                return carry3
            lax.fori_loop(0, _B, _row, 0)

            pltpu.sync_copy(enew_v, enew_out.at[c, pl.ds(row0, _B)])
            pltpu.sync_copy(val_v, val_out.at[c, pl.ds(row0, _B)])
            pltpu.sync_copy(val_v, acc_sh.at[dstp_v.at[j]], add=True)
            return carry2
        lax.fori_loop(0, _G, _batch1, 0)
        return carry
    lax.fori_loop(0, _NG, _group1, 0)

    plsc.subcore_barrier()
    _drain(0)
    plsc.subcore_barrier()
    _zero_acc()
    plsc.subcore_barrier()

    def _group2(g, carry):
        pltpu.sync_copy(dstb.at[s, g], dstp_v)

        def _batch2(j, carry2):
            row0 = s * _EPT + (g * _G + j) * _B
            pltpu.sync_copy(val_out.at[c, pl.ds(row0, _B)], val_v)
            pltpu.sync_copy(val_v, acc_sh.at[dstp_v.at[j]], add=True)
            return carry2
        lax.fori_loop(0, _G, _batch2, 0)
        return carry
    lax.fori_loop(0, _NG, _group2, 0)

    plsc.subcore_barrier()
    _drain(1)


def _sc_messages(db2, eh2, ce2, srcg, dstf, dsta, dstb):
    mesh = plsc.VectorSubcoreMesh(core_axis_name="c", subcore_axis_name="s")
    fn = functools.partial(
        pl.kernel, _sc_body, mesh=mesh,
        out_type=[
            jax.ShapeDtypeStruct((_NCORE, _E, _H), jnp.float32),
            jax.ShapeDtypeStruct((_NCORE, _E, _D), jnp.float32),
            jax.ShapeDtypeStruct((_NCORE, 2, _PHN, _D), jnp.float32),
        ],
        scratch_types=[
            pltpu.VMEM((_G, _B), jnp.int32),
            pltpu.VMEM((_G, _B), jnp.int32),
            pltpu.VMEM((_G, _B), jnp.int32),
            pltpu.VMEM((_B, _D), jnp.float32),
            pltpu.VMEM((_B, _D), jnp.float32),
            pltpu.VMEM((_B, _H), jnp.float32),
            pltpu.VMEM((_B, _H), jnp.float32),
            pltpu.VMEM((_B, _D), jnp.float32),
            pltpu.VMEM_SHARED((_PACC, _D), jnp.float32),
            pltpu.SemaphoreType.DMA,
            pltpu.SemaphoreType.DMA,
        ],
    )
    return fn()(db2, eh2, ce2, srcg, dstf, dsta, dstb)


def _estats_body(en_ref, out_ref, acc_ref):
    i = pl.program_id(0)

    @pl.when(i == 0)
    def _():
        acc_ref[...] = jnp.zeros_like(acc_ref)

    blk = jnp.concatenate([en_ref[0], en_ref[1]], axis=1)
    acc_ref[0:1] = acc_ref[0:1] + jnp.sum(blk, axis=0, keepdims=True)
    acc_ref[1:2] = acc_ref[1:2] + jnp.sum(blk * blk, axis=0, keepdims=True)

    @pl.when(i == _NBE - 1)
    def _():
        out_ref[...] = acc_ref[...]


def _estats(enew2):
    return pl.pallas_call(
        _estats_body,
        grid=(_NBE,),
        in_specs=[pl.BlockSpec((_NCORE, _BE, _H), lambda i: (0, i, 0))],
        out_specs=pl.BlockSpec((8, _D), lambda i: (0, 0)),
        out_shape=jax.ShapeDtypeStruct((8, _D), jnp.float32),
        scratch_shapes=[pltpu.VMEM((8, _D), jnp.float32)],
    )(enew2)


def _hpre_body(acc_in, ah_ref, hnp_ref, out_ref, st_ref):
    i = pl.program_id(0)

    @pl.when(i == 0)
    def _():
        st_ref[...] = jnp.zeros_like(st_ref)

    sh = jnp.concatenate([acc_in[0, :, :_H], acc_in[1, :, :_H]], axis=1)
    ss = jnp.concatenate([acc_in[0, :, _H:], acc_in[1, :, _H:]], axis=1)
    ah = jnp.concatenate([ah_ref[0], ah_ref[1]], axis=1)
    hnp = ah + sh / (ss + 1e-6)
    hnp_ref[...] = hnp
    st_ref[0:1] = st_ref[0:1] + jnp.sum(hnp, axis=0, keepdims=True)
    st_ref[1:2] = st_ref[1:2] + jnp.sum(hnp * hnp, axis=0, keepdims=True)

    @pl.when(i == _NBN - 1)
    def _():
        out_ref[...] = st_ref[...]


def _hpre(acc, Ah):
    return pl.pallas_call(
        _hpre_body,
        grid=(_NBN,),
        in_specs=[
            pl.BlockSpec((_NCORE, _BN_NODE, _D), lambda i: (0, i, 0)),
            pl.BlockSpec((_NCORE, _BN_NODE, _H), lambda i: (0, i, 0)),
        ],
        out_specs=[
            pl.BlockSpec((_BN_NODE, _D), lambda i: (i, 0)),
            pl.BlockSpec((8, _D), lambda i: (0, 0)),
        ],
        out_shape=[
            jax.ShapeDtypeStruct((_N, _D), jnp.float32),
            jax.ShapeDtypeStruct((8, _D), jnp.float32),
        ],
        scratch_shapes=[pltpu.VMEM((8, _D), jnp.float32)],
    )(acc, Ah)


def _bn_apply_body(nrows, x_ref, xn_ref, st_ref, g_ref, b_ref, out_ref):
    mu = st_ref[0:1] / nrows
    var = st_ref[1:2] / nrows - mu * mu
    xn = (xn_ref[...] - mu) * lax.rsqrt(var + 1e-5) * g_ref[...] + b_ref[...]
    out_ref[...] = x_ref[...] + jnp.maximum(xn, 0.0)


def _h_apply(h, hnp, hstats, g2, b2):
    return pl.pallas_call(
        functools.partial(_bn_apply_body, float(_N)),
        grid=(_NBN,),
        in_specs=[
            pl.BlockSpec((_BN_NODE, _D), lambda i: (i, 0)),
            pl.BlockSpec((_BN_NODE, _D), lambda i: (i, 0)),
            pl.BlockSpec((8, _D), lambda i: (0, 0)),
            pl.BlockSpec((1, _D), lambda i: (0, 0)),
            pl.BlockSpec((1, _D), lambda i: (0, 0)),
        ],
        out_specs=pl.BlockSpec((_BN_NODE, _D), lambda i: (i, 0)),
        out_shape=jax.ShapeDtypeStruct((_N, _D), jnp.float32),
    )(h, hnp, hstats, g2, b2)


def _e_apply_body(e_ref, en_ref, st_ref, g_ref, b_ref, out_ref):
    en = jnp.concatenate([en_ref[0], en_ref[1]], axis=1)
    mu = st_ref[0:1] / float(_E)
    var = st_ref[1:2] / float(_E) - mu * mu
    xn = (en - mu) * lax.rsqrt(var + 1e-5) * g_ref[...] + b_ref[...]
    out_ref[...] = e_ref[...] + jnp.maximum(xn, 0.0)


def _e_apply(e, enew2, estats, g2, b2):
    return pl.pallas_call(
        _e_apply_body,
        grid=(_NBE,),
        in_specs=[
            pl.BlockSpec((_BE, _D), lambda i: (i, 0)),
            pl.BlockSpec((_NCORE, _BE, _H), lambda i: (0, i, 0)),
            pl.BlockSpec((8, _D), lambda i: (0, 0)),
            pl.BlockSpec((1, _D), lambda i: (0, 0)),
            pl.BlockSpec((1, _D), lambda i: (0, 0)),
        ],
        out_specs=pl.BlockSpec((_BE, _D), lambda i: (i, 0)),
        out_shape=jax.ShapeDtypeStruct((_E, _D), jnp.float32),
    )(e, enew2, estats, g2, b2)


def kernel(h, e, edge_index, A_w, A_b, B_w, B_b, C_w, C_b, D_w, D_b, E_w, E_b,
           bn_h_gamma, bn_h_beta, bn_e_gamma, bn_e_beta):
    src = edge_index[0].reshape(_NSUB, _NG, _G, _B)
    dst = edge_index[1].reshape(_NSUB, _NG, _G, _B)
    srcg = jnp.stack([src, src + _N])
    dsta = jnp.where(dst < _PHN, dst, _PHN)
    dstb = jnp.where(dst >= _PHN, dst - _PHN, _PHN)

    r1 = lambda v: v.reshape(1, _D)
    w2 = lambda W: W.reshape(_D, _NCORE, _H).transpose(1, 0, 2)
    b2 = lambda v: v.reshape(_NCORE, 1, _H)
    Ah, db2, eh2 = _node_mm(h, w2(A_w), b2(A_b), w2(B_w), b2(B_b),
                            w2(D_w), b2(D_b), E_w, r1(E_b))
    ce2 = _edge_mm(e, w2(C_w), b2(C_b))
    enew2, _val, acc4 = _sc_messages(db2, eh2, ce2, srcg, dst, dsta, dstb)
    acc = acc4.reshape(_NCORE, 2 * _PHN, _D)
    estats = _estats(enew2)
    hnp, hstats = _hpre(acc, Ah)
    h_out = _h_apply(h, hnp, hstats, r1(bn_h_gamma), r1(bn_h_beta))
    e_out = _e_apply(e, enew2, estats, r1(bn_e_gamma), r1(bn_e_beta))
    return (h_out, e_out)


# speedup vs baseline: 1.3419x; 1.3419x over previous
import functools

import jax
import jax.numpy as jnp
from jax import lax
from jax.experimental import pallas as pl
from jax.experimental.pallas import tpu as pltpu
from jax.experimental.pallas import tpu_sc as plsc

_N = 10000
_E = 320000
_D = 128
_H = 64
_NSUB = 16
_NCORE = 2
_EPT = _E // _NSUB
_B = 40
_NB = _EPT // _B
_G = 10
_NG = _NB // _G

_BN_NODE = 400
_NBN = _N // _BN_NODE
_BE = 512
_NBE = _E // _BE


def _node_mm_body(h_ref, aw, ab, bw, bb, dw, db, ew, eb, ah_ref, db2_ref, eh2_ref):
    hblk = h_ref[...]
    ah_ref[0] = jnp.dot(hblk, aw[0], preferred_element_type=jnp.float32) + ab[0]
    dh = jnp.dot(hblk, dw[0], preferred_element_type=jnp.float32) + db[0]
    bh = jnp.dot(hblk, bw[0], preferred_element_type=jnp.float32) + bb[0]
    db2_ref[:, :_H] = dh
    db2_ref[:, _H:] = bh
    eh2_ref[...] = jnp.dot(hblk, ew[...], preferred_element_type=jnp.float32) + eb[...]


def _node_mm(h, A_w, A_b2, B_w, B_b2, D_w, D_b2, E_w, E_b2):
    wspec = pl.BlockSpec((1, _D, _H), lambda nb, c: (c, 0, 0))
    bspec = pl.BlockSpec((1, 1, _H), lambda nb, c: (c, 0, 0))
    return pl.pallas_call(
        _node_mm_body,
        grid=(_NBN, _NCORE),
        in_specs=[
            pl.BlockSpec((_BN_NODE, _D), lambda nb, c: (nb, 0)),
            wspec, bspec, wspec, bspec, wspec, bspec,
            pl.BlockSpec((_D, _D), lambda nb, c: (0, 0)),
            pl.BlockSpec((1, _D), lambda nb, c: (0, 0)),
        ],
        out_specs=[
            pl.BlockSpec((1, _BN_NODE, _H), lambda nb, c: (c, nb, 0)),
            pl.BlockSpec((_BN_NODE, _D), lambda nb, c: (c * _NBN + nb, 0)),
            pl.BlockSpec((_BN_NODE, _D), lambda nb, c: (nb, 0)),
        ],
        out_shape=[
            jax.ShapeDtypeStruct((_NCORE, _N, _H), jnp.float32),
            jax.ShapeDtypeStruct((2 * _N, _D), jnp.float32),
            jax.ShapeDtypeStruct((_N, _D), jnp.float32),
        ],
    )(h, A_w, A_b2, B_w, B_b2, D_w, D_b2, E_w, E_b2)


def _edge_mm_body(e_ref, cw, cb, ce_ref):
    ce_ref[0] = jnp.dot(e_ref[...], cw[0], preferred_element_type=jnp.float32) + cb[0]


def _edge_mm(e, C_w, C_b2):
    return pl.pallas_call(
        _edge_mm_body,
        grid=(_NBE, _NCORE),
        in_specs=[
            pl.BlockSpec((_BE, _D), lambda ebk, c: (ebk, 0)),
            pl.BlockSpec((1, _D, _H), lambda ebk, c: (c, 0, 0)),
            pl.BlockSpec((1, 1, _H), lambda ebk, c: (c, 0, 0)),
        ],
        out_specs=pl.BlockSpec((1, _BE, _H), lambda ebk, c: (c, ebk, 0)),
        out_shape=jax.ShapeDtypeStruct((_NCORE, _E, _H), jnp.float32),
    )(e, C_w, C_b2)


_PHN = 5120
_PACC = 5248
_ZPT = _PACC // _NSUB
_RPT = _PHN // _NSUB


def _sc_body(db2, eh2, ce2, srcg, dstf, dsta, dstb,
             enew_out, val_out, acc_out,
             srcg_v, dstf_v, dstp_v, db_v, eh_v, ce_v, enew_v, val_v,
             acc_sh, sdb0, sdb1, seh0, seh1, sce0, sce1,
             sen0, sen1, sva0, sva1):
    c = lax.axis_index("c")
    s = lax.axis_index("s")
    zero16 = jnp.zeros((16,), jnp.float32)
    co = c * _H
    sdb = (sdb0, sdb1)
    seh = (seh0, seh1)
    sce = (sce0, sce1)
    sen = (sen0, sen1)
    sva = (sva0, sva1)

    def _row0(g, jl):
        return s * _EPT + (g * _G + jl) * _B

    def _zero_acc():
        def _zrow(r, carry):
            for i in range(_D // 16):
                val_v[0, r, pl.ds(i * 16, 16)] = zero16
            return carry
        lax.fori_loop(0, _B, _zrow, 0)
        base = s * _ZPT
        for q in range(_ZPT // _B):
            pltpu.sync_copy(val_v.at[0], acc_sh.at[pl.ds(base + q * _B, _B)])
        rem = _ZPT - (_ZPT // _B) * _B
        pltpu.sync_copy(val_v.at[0, pl.ds(0, rem)],
                        acc_sh.at[pl.ds(base + (_ZPT // _B) * _B, rem)])

    def _drain(p):
        pltpu.sync_copy(acc_sh.at[pl.ds(s * _RPT, _RPT)],
                        acc_out.at[c, p, pl.ds(s * _RPT, _RPT)])

    def _load_idx1(g):
        pltpu.sync_copy(srcg.at[c, s, g], srcg_v)
        pltpu.sync_copy(dstf.at[s, g], dstf_v)
        pltpu.sync_copy(dsta.at[s, g], dstp_v)

    def _issue_in1(g, jl, b):
        pltpu.async_copy(db2.at[srcg_v.at[jl]], db_v.at[b], sdb[b])
        pltpu.async_copy(eh2.at[dstf_v.at[jl]], eh_v.at[b], seh[b])
        pltpu.async_copy(ce2.at[c, pl.ds(_row0(g, jl), _B)], ce_v.at[b], sce[b])

    def _wait_in1(b):
        pltpu.make_async_copy(db2.at[srcg_v.at[0]], db_v.at[b], sdb[b]).wait()
        pltpu.make_async_copy(eh2.at[dstf_v.at[0]], eh_v.at[b], seh[b]).wait()
        pltpu.make_async_copy(ce2.at[c, pl.ds(0, _B)], ce_v.at[b], sce[b]).wait()

    def _wait_writes(b):
        pltpu.make_async_copy(enew_v.at[b], enew_out.at[c, pl.ds(0, _B)],
                              sen[b]).wait()
        pltpu.make_async_copy(val_v.at[b], val_out.at[c, pl.ds(0, _B)],
                              sva[b]).wait()

    _zero_acc()
    plsc.subcore_barrier()

    _load_idx1(0)
    _issue_in1(0, 0, 0)

    def _compute(b):
        def _row(r, carry3):
            for i in range(_H // 16):
                o = i * 16
                d_ = db_v[b, r, pl.ds(o, 16)]
                b_ = db_v[b, r, pl.ds(_H + o, 16)]
                x = (d_ + eh_v[b, r, pl.ds(co + o, 16)]
                     + ce_v[b, r, pl.ds(o, 16)])
                enew_v[b, r, pl.ds(o, 16)] = x
                sg = 1.0 / (1.0 + jnp.exp(-x))
                val_v[b, r, pl.ds(o, 16)] = sg * b_
                val_v[b, r, pl.ds(_H + o, 16)] = sg
            return carry3
        lax.fori_loop(0, _B, _row, 0)

    def _group1(g, carry):
        for jl in range(_G):
            b = jl % 2
            row0 = _row0(g, jl)

            @pl.when(jnp.logical_or(g > 0, jl >= 2))
            def _():
                _wait_writes(b)
            _wait_in1(b)
            if jl + 1 < _G:
                _issue_in1(g, jl + 1, 1 - b)
            _compute(b)
            pltpu.async_copy(enew_v.at[b], enew_out.at[c, pl.ds(row0, _B)],
                             sen[b])
            pltpu.async_copy(val_v.at[b], val_out.at[c, pl.ds(row0, _B)],
                             sva[b])
            pltpu.sync_copy(val_v.at[b], acc_sh.at[dstp_v.at[jl]], add=True)

        @pl.when(g < _NG - 1)
        def _():
            _load_idx1(g + 1)
            _issue_in1(g + 1, 0, 0)
        return carry
    lax.fori_loop(0, _NG, _group1, 0)
    for b in range(2):
        _wait_writes(b)

    plsc.subcore_barrier()
    _drain(0)
    plsc.subcore_barrier()
    _zero_acc()
    plsc.subcore_barrier()

    def _issue_val(g, jl, b):
        pltpu.async_copy(val_out.at[c, pl.ds(_row0(g, jl), _B)], val_v.at[b],
                         sva[b])

    def _wait_val(b):
        pltpu.make_async_copy(val_out.at[c, pl.ds(0, _B)], val_v.at[b],
                              sva[b]).wait()

    pltpu.sync_copy(dstb.at[s, 0], dstp_v)
    _issue_val(0, 0, 0)

    def _group2(g, carry):
        for jl in range(_G):
            b = jl % 2
            _wait_val(b)
            if jl + 1 < _G:
                _issue_val(g, jl + 1, 1 - b)
            pltpu.sync_copy(val_v.at[b], acc_sh.at[dstp_v.at[jl]], add=True)

        @pl.when(g < _NG - 1)
        def _():
            pltpu.sync_copy(dstb.at[s, g + 1], dstp_v)
            _issue_val(g + 1, 0, 0)
        return carry
    lax.fori_loop(0, _NG, _group2, 0)

    plsc.subcore_barrier()
    _drain(1)


def _sc_messages(db2, eh2, ce2, srcg, dstf, dsta, dstb):
    mesh = plsc.VectorSubcoreMesh(core_axis_name="c", subcore_axis_name="s")
    fn = functools.partial(
        pl.kernel, _sc_body, mesh=mesh,
        out_type=[
            jax.ShapeDtypeStruct((_NCORE, _E, _H), jnp.float32),
            jax.ShapeDtypeStruct((_NCORE, _E, _D), jnp.float32),
            jax.ShapeDtypeStruct((_NCORE, 2, _PHN, _D), jnp.float32),
        ],
        scratch_types=[
            pltpu.VMEM((_G, _B), jnp.int32),
            pltpu.VMEM((_G, _B), jnp.int32),
            pltpu.VMEM((_G, _B), jnp.int32),
            pltpu.VMEM((2, _B, _D), jnp.float32),
            pltpu.VMEM((2, _B, _D), jnp.float32),
            pltpu.VMEM((2, _B, _H), jnp.float32),
            pltpu.VMEM((2, _B, _H), jnp.float32),
            pltpu.VMEM((2, _B, _D), jnp.float32),
            pltpu.VMEM_SHARED((_PACC, _D), jnp.float32),
        ] + [pltpu.SemaphoreType.DMA] * 10,
    )
    return fn()(db2, eh2, ce2, srcg, dstf, dsta, dstb)


def _estats_body(en_ref, out_ref, acc_ref):
    i = pl.program_id(0)

    @pl.when(i == 0)
    def _():
        acc_ref[...] = jnp.zeros_like(acc_ref)

    blk = jnp.concatenate([en_ref[0], en_ref[1]], axis=1)
    acc_ref[0:1] = acc_ref[0:1] + jnp.sum(blk, axis=0, keepdims=True)
    acc_ref[1:2] = acc_ref[1:2] + jnp.sum(blk * blk, axis=0, keepdims=True)

    @pl.when(i == _NBE - 1)
    def _():
        out_ref[...] = acc_ref[...]


def _estats(enew2):
    return pl.pallas_call(
        _estats_body,
        grid=(_NBE,),
        in_specs=[pl.BlockSpec((_NCORE, _BE, _H), lambda i: (0, i, 0))],
        out_specs=pl.BlockSpec((8, _D), lambda i: (0, 0)),
        out_shape=jax.ShapeDtypeStruct((8, _D), jnp.float32),
        scratch_shapes=[pltpu.VMEM((8, _D), jnp.float32)],
    )(enew2)


def _hpre_body(acc_in, ah_ref, hnp_ref, out_ref, st_ref):
    i = pl.program_id(0)

    @pl.when(i == 0)
    def _():
        st_ref[...] = jnp.zeros_like(st_ref)

    sh = jnp.concatenate([acc_in[0, :, :_H], acc_in[1, :, :_H]], axis=1)
    ss = jnp.concatenate([acc_in[0, :, _H:], acc_in[1, :, _H:]], axis=1)
    ah = jnp.concatenate([ah_ref[0], ah_ref[1]], axis=1)
    hnp = ah + sh / (ss + 1e-6)
    hnp_ref[...] = hnp
    st_ref[0:1] = st_ref[0:1] + jnp.sum(hnp, axis=0, keepdims=True)
    st_ref[1:2] = st_ref[1:2] + jnp.sum(hnp * hnp, axis=0, keepdims=True)

    @pl.when(i == _NBN - 1)
    def _():
        out_ref[...] = st_ref[...]


def _hpre(acc, Ah):
    return pl.pallas_call(
        _hpre_body,
        grid=(_NBN,),
        in_specs=[
            pl.BlockSpec((_NCORE, _BN_NODE, _D), lambda i: (0, i, 0)),
            pl.BlockSpec((_NCORE, _BN_NODE, _H), lambda i: (0, i, 0)),
        ],
        out_specs=[
            pl.BlockSpec((_BN_NODE, _D), lambda i: (i, 0)),
            pl.BlockSpec((8, _D), lambda i: (0, 0)),
        ],
        out_shape=[
            jax.ShapeDtypeStruct((_N, _D), jnp.float32),
            jax.ShapeDtypeStruct((8, _D), jnp.float32),
        ],
        scratch_shapes=[pltpu.VMEM((8, _D), jnp.float32)],
    )(acc, Ah)


def _bn_apply_body(nrows, x_ref, xn_ref, st_ref, g_ref, b_ref, out_ref):
    mu = st_ref[0:1] / nrows
    var = st_ref[1:2] / nrows - mu * mu
    xn = (xn_ref[...] - mu) * lax.rsqrt(var + 1e-5) * g_ref[...] + b_ref[...]
    out_ref[...] = x_ref[...] + jnp.maximum(xn, 0.0)


def _h_apply(h, hnp, hstats, g2, b2):
    return pl.pallas_call(
        functools.partial(_bn_apply_body, float(_N)),
        grid=(_NBN,),
        in_specs=[
            pl.BlockSpec((_BN_NODE, _D), lambda i: (i, 0)),
            pl.BlockSpec((_BN_NODE, _D), lambda i: (i, 0)),
            pl.BlockSpec((8, _D), lambda i: (0, 0)),
            pl.BlockSpec((1, _D), lambda i: (0, 0)),
            pl.BlockSpec((1, _D), lambda i: (0, 0)),
        ],
        out_specs=pl.BlockSpec((_BN_NODE, _D), lambda i: (i, 0)),
        out_shape=jax.ShapeDtypeStruct((_N, _D), jnp.float32),
    )(h, hnp, hstats, g2, b2)


def _e_apply_body(e_ref, en_ref, st_ref, g_ref, b_ref, out_ref):
    en = jnp.concatenate([en_ref[0], en_ref[1]], axis=1)
    mu = st_ref[0:1] / float(_E)
    var = st_ref[1:2] / float(_E) - mu * mu
    xn = (en - mu) * lax.rsqrt(var + 1e-5) * g_ref[...] + b_ref[...]
    out_ref[...] = e_ref[...] + jnp.maximum(xn, 0.0)


def _e_apply(e, enew2, estats, g2, b2):
    return pl.pallas_call(
        _e_apply_body,
        grid=(_NBE,),
        in_specs=[
            pl.BlockSpec((_BE, _D), lambda i: (i, 0)),
            pl.BlockSpec((_NCORE, _BE, _H), lambda i: (0, i, 0)),
            pl.BlockSpec((8, _D), lambda i: (0, 0)),
            pl.BlockSpec((1, _D), lambda i: (0, 0)),
            pl.BlockSpec((1, _D), lambda i: (0, 0)),
        ],
        out_specs=pl.BlockSpec((_BE, _D), lambda i: (i, 0)),
        out_shape=jax.ShapeDtypeStruct((_E, _D), jnp.float32),
    )(e, enew2, estats, g2, b2)


def kernel(h, e, edge_index, A_w, A_b, B_w, B_b, C_w, C_b, D_w, D_b, E_w, E_b,
           bn_h_gamma, bn_h_beta, bn_e_gamma, bn_e_beta):
    src = edge_index[0].reshape(_NSUB, _NG, _G, _B)
    dst = edge_index[1].reshape(_NSUB, _NG, _G, _B)
    srcg = jnp.stack([src, src + _N])
    dsta = jnp.where(dst < _PHN, dst, _PHN)
    dstb = jnp.where(dst >= _PHN, dst - _PHN, _PHN)

    r1 = lambda v: v.reshape(1, _D)
    w2 = lambda W: W.reshape(_D, _NCORE, _H).transpose(1, 0, 2)
    b2 = lambda v: v.reshape(_NCORE, 1, _H)
    Ah, db2, eh2 = _node_mm(h, w2(A_w), b2(A_b), w2(B_w), b2(B_b),
                            w2(D_w), b2(D_b), E_w, r1(E_b))
    ce2 = _edge_mm(e, w2(C_w), b2(C_b))
    enew2, _val, acc4 = _sc_messages(db2, eh2, ce2, srcg, dst, dsta, dstb)
    acc = acc4.reshape(_NCORE, 2 * _PHN, _D)
    estats = _estats(enew2)
    hnp, hstats = _hpre(acc, Ah)
    h_out = _h_apply(h, hnp, hstats, r1(bn_h_gamma), r1(bn_h_beta))
    e_out = _e_apply(e, enew2, estats, r1(bn_e_gamma), r1(bn_e_beta))
    return (h_out, e_out)


# speedup vs baseline: 1.8696x; 1.3933x over previous
import functools

import jax
import jax.numpy as jnp
from jax import lax
from jax.experimental import pallas as pl
from jax.experimental.pallas import tpu as pltpu
from jax.experimental.pallas import tpu_sc as plsc

_N = 10000
_E = 320000
_D = 128
_H = 64
_NSUB = 16
_NCORE = 2
_EPT = _E // _NSUB
_B = 40
_NB = _EPT // _B
_G = 10
_NG = _NB // _G

_BN_NODE = 2000
_NBN = _N // _BN_NODE
_BE = 2560
_NBE = _E // _BE


def _node_mm_body(h_ref, aw, ab, bw, bb, dw, db, ew, eb, ah_ref, db2_ref, eh2_ref):
    hblk = h_ref[...]
    ah_ref[0] = jnp.dot(hblk, aw[0], preferred_element_type=jnp.float32) + ab[0]
    dh = jnp.dot(hblk, dw[0], preferred_element_type=jnp.float32) + db[0]
    bh = jnp.dot(hblk, bw[0], preferred_element_type=jnp.float32) + bb[0]
    db2_ref[:, :_H] = dh
    db2_ref[:, _H:] = bh
    eh2_ref[...] = jnp.dot(hblk, ew[...], preferred_element_type=jnp.float32) + eb[...]


def _node_mm(h, A_w, A_b2, B_w, B_b2, D_w, D_b2, E_w, E_b2):
    wspec = pl.BlockSpec((1, _D, _H), lambda nb, c: (c, 0, 0))
    bspec = pl.BlockSpec((1, 1, _H), lambda nb, c: (c, 0, 0))
    return pl.pallas_call(
        _node_mm_body,
        grid=(_NBN, _NCORE),
        in_specs=[
            pl.BlockSpec((_BN_NODE, _D), lambda nb, c: (nb, 0)),
            wspec, bspec, wspec, bspec, wspec, bspec,
            pl.BlockSpec((_D, _D), lambda nb, c: (0, 0)),
            pl.BlockSpec((1, _D), lambda nb, c: (0, 0)),
        ],
        out_specs=[
            pl.BlockSpec((1, _BN_NODE, _H), lambda nb, c: (c, nb, 0)),
            pl.BlockSpec((_BN_NODE, _D), lambda nb, c: (c * _NBN + nb, 0)),
            pl.BlockSpec((_BN_NODE, _D), lambda nb, c: (nb, 0)),
        ],
        out_shape=[
            jax.ShapeDtypeStruct((_NCORE, _N, _H), jnp.float32),
            jax.ShapeDtypeStruct((2 * _N, _D), jnp.float32),
            jax.ShapeDtypeStruct((_N, _D), jnp.float32),
        ],
    )(h, A_w, A_b2, B_w, B_b2, D_w, D_b2, E_w, E_b2)


def _edge_mm_body(e_ref, cw, cb, ce_ref):
    ce_ref[0] = jnp.dot(e_ref[...], cw[0], preferred_element_type=jnp.float32) + cb[0]


def _edge_mm(e, C_w, C_b2):
    return pl.pallas_call(
        _edge_mm_body,
        grid=(_NBE, _NCORE),
        in_specs=[
            pl.BlockSpec((_BE, _D), lambda ebk, c: (ebk, 0)),
            pl.BlockSpec((1, _D, _H), lambda ebk, c: (c, 0, 0)),
            pl.BlockSpec((1, 1, _H), lambda ebk, c: (c, 0, 0)),
        ],
        out_specs=pl.BlockSpec((1, _BE, _H), lambda ebk, c: (c, ebk, 0)),
        out_shape=jax.ShapeDtypeStruct((_NCORE, _E, _H), jnp.float32),
    )(e, C_w, C_b2)


_PHN = 5120
_PACC = 5248
_ZPT = _PACC // _NSUB
_RPT = _PHN // _NSUB


def _sc_body(db2, eh2, ce2, srcg, dstf, dsta, dstb,
             enew_out, val_out, acc_out,
             srcg_v, dstf_v, dstp_v, db_v, eh_v, ce_v, enew_v, val_v,
             acc_sh, sdb0, sdb1, seh0, seh1, sce0, sce1,
             sen0, sen1, sva0, sva1, ssc0, ssc1):
    c = lax.axis_index("c")
    s = lax.axis_index("s")
    zero16 = jnp.zeros((16,), jnp.float32)
    co = c * _H
    sdb = (sdb0, sdb1)
    seh = (seh0, seh1)
    sce = (sce0, sce1)
    sen = (sen0, sen1)
    sva = (sva0, sva1)
    ssc = (ssc0, ssc1)

    def _row0(g, jl):
        return s * _EPT + (g * _G + jl) * _B

    def _zero_acc():
        def _zrow(r, carry):
            for i in range(_D // 16):
                val_v[0, r, pl.ds(i * 16, 16)] = zero16
            return carry
        lax.fori_loop(0, _B, _zrow, 0)
        base = s * _ZPT
        for q in range(_ZPT // _B):
            pltpu.sync_copy(val_v.at[0], acc_sh.at[pl.ds(base + q * _B, _B)])
        rem = _ZPT - (_ZPT // _B) * _B
        pltpu.sync_copy(val_v.at[0, pl.ds(0, rem)],
                        acc_sh.at[pl.ds(base + (_ZPT // _B) * _B, rem)])

    def _drain(p):
        pltpu.sync_copy(acc_sh.at[pl.ds(s * _RPT, _RPT)],
                        acc_out.at[c, p, pl.ds(s * _RPT, _RPT)])

    def _load_idx1(g):
        pltpu.sync_copy(srcg.at[c, s, g], srcg_v)
        pltpu.sync_copy(dstf.at[s, g], dstf_v)
        pltpu.sync_copy(dsta.at[s, g], dstp_v)

    def _issue_in1(g, jl, b):
        pltpu.async_copy(db2.at[srcg_v.at[jl]], db_v.at[b], sdb[b])
        pltpu.async_copy(eh2.at[dstf_v.at[jl]], eh_v.at[b], seh[b])
        pltpu.async_copy(ce2.at[c, pl.ds(_row0(g, jl), _B)], ce_v.at[b], sce[b])

    def _wait_in1(b):
        pltpu.make_async_copy(db2.at[srcg_v.at[0]], db_v.at[b], sdb[b]).wait()
        pltpu.make_async_copy(eh2.at[dstf_v.at[0]], eh_v.at[b], seh[b]).wait()
        pltpu.make_async_copy(ce2.at[c, pl.ds(0, _B)], ce_v.at[b], sce[b]).wait()

    def _wait_writes(b):
        pltpu.make_async_copy(enew_v.at[b], enew_out.at[c, pl.ds(0, _B)],
                              sen[b]).wait()
        pltpu.make_async_copy(val_v.at[b], val_out.at[c, pl.ds(0, _B)],
                              sva[b]).wait()
        pltpu.make_async_copy(val_v.at[b], acc_sh.at[dstp_v.at[0]],
                              ssc[b]).wait()

    _zero_acc()
    plsc.subcore_barrier()

    _load_idx1(0)
    _issue_in1(0, 0, 0)

    def _compute(b):
        def _row(r2, carry3):
            for u in range(2):
                r = r2 * 2 + u
                for i in range(_H // 16):
                    o = i * 16
                    d_ = db_v[b, r, pl.ds(o, 16)]
                    b_ = db_v[b, r, pl.ds(_H + o, 16)]
                    x = (d_ + eh_v[b, r, pl.ds(co + o, 16)]
                         + ce_v[b, r, pl.ds(o, 16)])
                    enew_v[b, r, pl.ds(o, 16)] = x
                    sg = 1.0 / (1.0 + jnp.exp(-x))
                    val_v[b, r, pl.ds(o, 16)] = sg * b_
                    val_v[b, r, pl.ds(_H + o, 16)] = sg
            return carry3
        lax.fori_loop(0, _B // 2, _row, 0)

    def _group1(g, carry):
        for jl in range(_G):
            b = jl % 2
            row0 = _row0(g, jl)

            @pl.when(jnp.logical_or(g > 0, jl >= 2))
            def _():
                _wait_writes(b)
            _wait_in1(b)
            if jl + 1 < _G:
                _issue_in1(g, jl + 1, 1 - b)
            _compute(b)
            pltpu.async_copy(enew_v.at[b], enew_out.at[c, pl.ds(row0, _B)],
                             sen[b])
            pltpu.async_copy(val_v.at[b], val_out.at[c, pl.ds(row0, _B)],
                             sva[b])
            pltpu.async_copy(val_v.at[b], acc_sh.at[dstp_v.at[jl]], ssc[b],
                             add=True)

        @pl.when(g < _NG - 1)
        def _():
            _load_idx1(g + 1)
            _issue_in1(g + 1, 0, 0)
        return carry
    lax.fori_loop(0, _NG, _group1, 0)
    for b in range(2):
        _wait_writes(b)

    plsc.subcore_barrier()
    _drain(0)
    plsc.subcore_barrier()
    _zero_acc()
    plsc.subcore_barrier()

    def _issue_val(g, jl, b):
        pltpu.async_copy(val_out.at[c, pl.ds(_row0(g, jl), _B)], val_v.at[b],
                         sva[b])

    def _wait_val(b):
        pltpu.make_async_copy(val_out.at[c, pl.ds(0, _B)], val_v.at[b],
                              sva[b]).wait()

    pltpu.sync_copy(dstb.at[s, 0], dstp_v)
    _issue_val(0, 0, 0)

    def _wait_sc(b):
        pltpu.make_async_copy(val_v.at[b], acc_sh.at[dstp_v.at[0]],
                              ssc[b]).wait()

    def _group2(g, carry):
        for jl in range(_G):
            b = jl % 2

            @pl.when(jnp.logical_or(g > 0, jl >= 2))
            def _():
                _wait_sc(b)
            _wait_val(b)
            if jl + 1 < _G:
                _issue_val(g, jl + 1, 1 - b)
            pltpu.async_copy(val_v.at[b], acc_sh.at[dstp_v.at[jl]], ssc[b],
                             add=True)

        @pl.when(g < _NG - 1)
        def _():
            pltpu.sync_copy(dstb.at[s, g + 1], dstp_v)
            _issue_val(g + 1, 0, 0)
        return carry
    lax.fori_loop(0, _NG, _group2, 0)
    for b in range(2):
        _wait_sc(b)

    plsc.subcore_barrier()
    _drain(1)


def _sc_messages(db2, eh2, ce2, srcg, dstf, dsta, dstb):
    mesh = plsc.VectorSubcoreMesh(core_axis_name="c", subcore_axis_name="s")
    fn = functools.partial(
        pl.kernel, _sc_body, mesh=mesh,
        out_type=[
            jax.ShapeDtypeStruct((_NCORE, _E, _H), jnp.float32),
            jax.ShapeDtypeStruct((_NCORE, _E, _D), jnp.float32),
            jax.ShapeDtypeStruct((_NCORE, 2, _PHN, _D), jnp.float32),
        ],
        scratch_types=[
            pltpu.VMEM((_G, _B), jnp.int32),
            pltpu.VMEM((_G, _B), jnp.int32),
            pltpu.VMEM((_G, _B), jnp.int32),
            pltpu.VMEM((2, _B, _D), jnp.float32),
            pltpu.VMEM((2, _B, _D), jnp.float32),
            pltpu.VMEM((2, _B, _H), jnp.float32),
            pltpu.VMEM((2, _B, _H), jnp.float32),
            pltpu.VMEM((2, _B, _D), jnp.float32),
            pltpu.VMEM_SHARED((_PACC, _D), jnp.float32),
        ] + [pltpu.SemaphoreType.DMA] * 12,
    )
    return fn()(db2, eh2, ce2, srcg, dstf, dsta, dstb)


def _estats_body(en_ref, out_ref, acc_ref):
    i = pl.program_id(0)

    @pl.when(i == 0)
    def _():
        acc_ref[...] = jnp.zeros_like(acc_ref)

    blk = jnp.concatenate([en_ref[0], en_ref[1]], axis=1)
    acc_ref[0:1] = acc_ref[0:1] + jnp.sum(blk, axis=0, keepdims=True)
    acc_ref[1:2] = acc_ref[1:2] + jnp.sum(blk * blk, axis=0, keepdims=True)

    @pl.when(i == _NBE - 1)
    def _():
        out_ref[...] = acc_ref[...]


def _estats(enew2):
    return pl.pallas_call(
        _estats_body,
        grid=(_NBE,),
        in_specs=[pl.BlockSpec((_NCORE, _BE, _H), lambda i: (0, i, 0))],
        out_specs=pl.BlockSpec((8, _D), lambda i: (0, 0)),
        out_shape=jax.ShapeDtypeStruct((8, _D), jnp.float32),
        scratch_shapes=[pltpu.VMEM((8, _D), jnp.float32)],
    )(enew2)


def _hpre_body(acc_in, ah_ref, hnp_ref, out_ref, st_ref):
    i = pl.program_id(0)

    @pl.when(i == 0)
    def _():
        st_ref[...] = jnp.zeros_like(st_ref)

    sh = jnp.concatenate([acc_in[0, :, :_H], acc_in[1, :, :_H]], axis=1)
    ss = jnp.concatenate([acc_in[0, :, _H:], acc_in[1, :, _H:]], axis=1)
    ah = jnp.concatenate([ah_ref[0], ah_ref[1]], axis=1)
    hnp = ah + sh / (ss + 1e-6)
    hnp_ref[...] = hnp
    st_ref[0:1] = st_ref[0:1] + jnp.sum(hnp, axis=0, keepdims=True)
    st_ref[1:2] = st_ref[1:2] + jnp.sum(hnp * hnp, axis=0, keepdims=True)

    @pl.when(i == _NBN - 1)
    def _():
        out_ref[...] = st_ref[...]


def _hpre(acc, Ah):
    return pl.pallas_call(
        _hpre_body,
        grid=(_NBN,),
        in_specs=[
            pl.BlockSpec((_NCORE, _BN_NODE, _D), lambda i: (0, i, 0)),
            pl.BlockSpec((_NCORE, _BN_NODE, _H), lambda i: (0, i, 0)),
        ],
        out_specs=[
            pl.BlockSpec((_BN_NODE, _D), lambda i: (i, 0)),
            pl.BlockSpec((8, _D), lambda i: (0, 0)),
        ],
        out_shape=[
            jax.ShapeDtypeStruct((_N, _D), jnp.float32),
            jax.ShapeDtypeStruct((8, _D), jnp.float32),
        ],
        scratch_shapes=[pltpu.VMEM((8, _D), jnp.float32)],
    )(acc, Ah)


def _bn_apply_body(nrows, x_ref, xn_ref, st_ref, g_ref, b_ref, out_ref):
    mu = st_ref[0:1] / nrows
    var = st_ref[1:2] / nrows - mu * mu
    xn = (xn_ref[...] - mu) * lax.rsqrt(var + 1e-5) * g_ref[...] + b_ref[...]
    out_ref[...] = x_ref[...] + jnp.maximum(xn, 0.0)


def _h_apply(h, hnp, hstats, g2, b2):
    return pl.pallas_call(
        functools.partial(_bn_apply_body, float(_N)),
        grid=(_NBN,),
        in_specs=[
            pl.BlockSpec((_BN_NODE, _D), lambda i: (i, 0)),
            pl.BlockSpec((_BN_NODE, _D), lambda i: (i, 0)),
            pl.BlockSpec((8, _D), lambda i: (0, 0)),
            pl.BlockSpec((1, _D), lambda i: (0, 0)),
            pl.BlockSpec((1, _D), lambda i: (0, 0)),
        ],
        out_specs=pl.BlockSpec((_BN_NODE, _D), lambda i: (i, 0)),
        out_shape=jax.ShapeDtypeStruct((_N, _D), jnp.float32),
    )(h, hnp, hstats, g2, b2)


def _e_apply_body(e_ref, en_ref, st_ref, g_ref, b_ref, out_ref):
    en = jnp.concatenate([en_ref[0], en_ref[1]], axis=1)
    mu = st_ref[0:1] / float(_E)
    var = st_ref[1:2] / float(_E) - mu * mu
    xn = (en - mu) * lax.rsqrt(var + 1e-5) * g_ref[...] + b_ref[...]
    out_ref[...] = e_ref[...] + jnp.maximum(xn, 0.0)


def _e_apply(e, enew2, estats, g2, b2):
    return pl.pallas_call(
        _e_apply_body,
        grid=(_NBE,),
        in_specs=[
            pl.BlockSpec((_BE, _D), lambda i: (i, 0)),
            pl.BlockSpec((_NCORE, _BE, _H), lambda i: (0, i, 0)),
            pl.BlockSpec((8, _D), lambda i: (0, 0)),
            pl.BlockSpec((1, _D), lambda i: (0, 0)),
            pl.BlockSpec((1, _D), lambda i: (0, 0)),
        ],
        out_specs=pl.BlockSpec((_BE, _D), lambda i: (i, 0)),
        out_shape=jax.ShapeDtypeStruct((_E, _D), jnp.float32),
    )(e, enew2, estats, g2, b2)


def kernel(h, e, edge_index, A_w, A_b, B_w, B_b, C_w, C_b, D_w, D_b, E_w, E_b,
           bn_h_gamma, bn_h_beta, bn_e_gamma, bn_e_beta):
    src = edge_index[0].reshape(_NSUB, _NG, _G, _B)
    dst = edge_index[1].reshape(_NSUB, _NG, _G, _B)
    srcg = jnp.stack([src, src + _N])
    dsta = jnp.where(dst < _PHN, dst, _PHN)
    dstb = jnp.where(dst >= _PHN, dst - _PHN, _PHN)

    r1 = lambda v: v.reshape(1, _D)
    w2 = lambda W: W.reshape(_D, _NCORE, _H).transpose(1, 0, 2)
    b2 = lambda v: v.reshape(_NCORE, 1, _H)
    Ah, db2, eh2 = _node_mm(h, w2(A_w), b2(A_b), w2(B_w), b2(B_b),
                            w2(D_w), b2(D_b), E_w, r1(E_b))
    ce2 = _edge_mm(e, w2(C_w), b2(C_b))
    enew2, _val, acc4 = _sc_messages(db2, eh2, ce2, srcg, dst, dsta, dstb)
    acc = acc4.reshape(_NCORE, 2 * _PHN, _D)
    estats = _estats(enew2)
    hnp, hstats = _hpre(acc, Ah)
    h_out = _h_apply(h, hnp, hstats, r1(bn_h_gamma), r1(bn_h_beta))
    e_out = _e_apply(e, enew2, estats, r1(bn_e_gamma), r1(bn_e_beta))
    return (h_out, e_out)


# speedup vs baseline: 2.1251x; 1.1366x over previous
import functools

import jax
import jax.numpy as jnp
from jax import lax
from jax.experimental import pallas as pl
from jax.experimental.pallas import tpu as pltpu
from jax.experimental.pallas import tpu_sc as plsc

_N = 10000
_E = 320000
_D = 128
_H = 64
_NSUB = 16
_NCORE = 2
_EPT = _E // _NSUB
_B = 40
_NB = _EPT // _B
_G = 10
_NG = _NB // _G

_BN_NODE = 2000
_NBN = _N // _BN_NODE
_BE = 2560
_NBE = _E // _BE


def _node_mm_body(h_ref, aw, ab, bw, bb, dw, db, ew, eb, ah_ref, db2_ref, eh2_ref):
    hblk = h_ref[...]
    ah_ref[0] = jnp.dot(hblk, aw[0], preferred_element_type=jnp.float32) + ab[0]
    dh = jnp.dot(hblk, dw[0], preferred_element_type=jnp.float32) + db[0]
    bh = jnp.dot(hblk, bw[0], preferred_element_type=jnp.float32) + bb[0]
    db2_ref[:, :_H] = dh
    db2_ref[:, _H:] = bh
    eh2_ref[...] = jnp.dot(hblk, ew[...], preferred_element_type=jnp.float32) + eb[...]


def _node_mm(h, A_w, A_b2, B_w, B_b2, D_w, D_b2, E_w, E_b2):
    wspec = pl.BlockSpec((1, _D, _H), lambda nb, c: (c, 0, 0))
    bspec = pl.BlockSpec((1, 1, _H), lambda nb, c: (c, 0, 0))
    return pl.pallas_call(
        _node_mm_body,
        grid=(_NBN, _NCORE),
        in_specs=[
            pl.BlockSpec((_BN_NODE, _D), lambda nb, c: (nb, 0)),
            wspec, bspec, wspec, bspec, wspec, bspec,
            pl.BlockSpec((_D, _D), lambda nb, c: (0, 0)),
            pl.BlockSpec((1, _D), lambda nb, c: (0, 0)),
        ],
        out_specs=[
            pl.BlockSpec((1, _BN_NODE, _H), lambda nb, c: (c, nb, 0)),
            pl.BlockSpec((_BN_NODE, _D), lambda nb, c: (c * _NBN + nb, 0)),
            pl.BlockSpec((_BN_NODE, _D), lambda nb, c: (nb, 0)),
        ],
        out_shape=[
            jax.ShapeDtypeStruct((_NCORE, _N, _H), jnp.float32),
            jax.ShapeDtypeStruct((2 * _N, _D), jnp.float32),
            jax.ShapeDtypeStruct((_N, _D), jnp.float32),
        ],
    )(h, A_w, A_b2, B_w, B_b2, D_w, D_b2, E_w, E_b2)


def _edge_mm_body(e_ref, cw, cb, ce_ref):
    ce_ref[0] = jnp.dot(e_ref[...], cw[0], preferred_element_type=jnp.float32) + cb[0]


def _edge_mm(e, C_w, C_b2):
    return pl.pallas_call(
        _edge_mm_body,
        grid=(_NBE, _NCORE),
        in_specs=[
            pl.BlockSpec((_BE, _D), lambda ebk, c: (ebk, 0)),
            pl.BlockSpec((1, _D, _H), lambda ebk, c: (c, 0, 0)),
            pl.BlockSpec((1, 1, _H), lambda ebk, c: (c, 0, 0)),
        ],
        out_specs=pl.BlockSpec((1, _BE, _H), lambda ebk, c: (c, ebk, 0)),
        out_shape=jax.ShapeDtypeStruct((_NCORE, _E, _H), jnp.float32),
    )(e, C_w, C_b2)


_PHN = 5120
_PACC = 5248
_ZPT = _PACC // _NSUB
_RPT = _PHN // _NSUB


_B2 = 160
_NB2 = _EPT // _B2
_G2 = 5
_NG2 = _NB2 // _G2


def _sc1_body(db2, eh2, ce2, srcg, dstf, dsta,
              enew_out, val_out, acc_out,
              srcg_v, dstf_v, dstp_v, db_v, eh_v, ce_v, enew_v, val_v,
              acc_sh, sdb0, sdb1, seh0, seh1, sce0, sce1,
              sen0, sen1, sva0, sva1, ssc0, ssc1):
    c = lax.axis_index("c")
    s = lax.axis_index("s")
    zero16 = jnp.zeros((16,), jnp.float32)
    co = c * _H
    sdb = (sdb0, sdb1)
    seh = (seh0, seh1)
    sce = (sce0, sce1)
    sen = (sen0, sen1)
    sva = (sva0, sva1)
    ssc = (ssc0, ssc1)

    def _row0(g, jl):
        return s * _EPT + (g * _G + jl) * _B

    def _zrow(r, carry):
        for i in range(_D // 16):
            val_v[0, r, pl.ds(i * 16, 16)] = zero16
        return carry
    lax.fori_loop(0, _B, _zrow, 0)
    base = s * _ZPT
    for q in range(_ZPT // _B):
        pltpu.sync_copy(val_v.at[0], acc_sh.at[pl.ds(base + q * _B, _B)])
    rem = _ZPT - (_ZPT // _B) * _B
    pltpu.sync_copy(val_v.at[0, pl.ds(0, rem)],
                    acc_sh.at[pl.ds(base + (_ZPT // _B) * _B, rem)])
    plsc.subcore_barrier()

    def _load_idx(g):
        pltpu.sync_copy(srcg.at[c, s, g], srcg_v)
        pltpu.sync_copy(dstf.at[s, g], dstf_v)
        pltpu.sync_copy(dsta.at[s, g], dstp_v)

    def _issue_in(g, jl, b):
        pltpu.async_copy(db2.at[srcg_v.at[jl]], db_v.at[b], sdb[b])
        pltpu.async_copy(eh2.at[dstf_v.at[jl]], eh_v.at[b], seh[b])
        pltpu.async_copy(ce2.at[c, pl.ds(_row0(g, jl), _B)], ce_v.at[b], sce[b])

    def _wait_in(b):
        pltpu.make_async_copy(db2.at[srcg_v.at[0]], db_v.at[b], sdb[b]).wait()
        pltpu.make_async_copy(eh2.at[dstf_v.at[0]], eh_v.at[b], seh[b]).wait()
        pltpu.make_async_copy(ce2.at[c, pl.ds(0, _B)], ce_v.at[b], sce[b]).wait()

    def _wait_writes(b):
        pltpu.make_async_copy(enew_v.at[b], enew_out.at[c, pl.ds(0, _B)],
                              sen[b]).wait()
        pltpu.make_async_copy(val_v.at[b], val_out.at[c, pl.ds(0, _B)],
                              sva[b]).wait()
        pltpu.make_async_copy(val_v.at[b], acc_sh.at[dstp_v.at[0]],
                              ssc[b]).wait()

    def _compute(b):
        def _row(r2, carry3):
            for u in range(2):
                r = r2 * 2 + u
                for i in range(_H // 16):
                    o = i * 16
                    d_ = db_v[b, r, pl.ds(o, 16)]
                    b_ = db_v[b, r, pl.ds(_H + o, 16)]
                    x = (d_ + eh_v[b, r, pl.ds(co + o, 16)]
                         + ce_v[b, r, pl.ds(o, 16)])
                    enew_v[b, r, pl.ds(o, 16)] = x
                    sg = 1.0 / (1.0 + jnp.exp(-x))
                    val_v[b, r, pl.ds(o, 16)] = sg * b_
                    val_v[b, r, pl.ds(_H + o, 16)] = sg
            return carry3
        lax.fori_loop(0, _B // 2, _row, 0)

    _load_idx(0)
    _issue_in(0, 0, 0)

    def _group(g, carry):
        for jl in range(_G):
            b = jl % 2
            row0 = _row0(g, jl)

            @pl.when(jnp.logical_or(g > 0, jl >= 2))
            def _():
                _wait_writes(b)
            _wait_in(b)
            if jl + 1 < _G:
                _issue_in(g, jl + 1, 1 - b)
            _compute(b)
            pltpu.async_copy(enew_v.at[b], enew_out.at[c, pl.ds(row0, _B)],
                             sen[b])
            pltpu.async_copy(val_v.at[b], val_out.at[c, pl.ds(row0, _B)],
                             sva[b])
            pltpu.async_copy(val_v.at[b], acc_sh.at[dstp_v.at[jl]], ssc[b],
                             add=True)

        @pl.when(g < _NG - 1)
        def _():
            _load_idx(g + 1)
            _issue_in(g + 1, 0, 0)
        return carry
    lax.fori_loop(0, _NG, _group, 0)
    for b in range(2):
        _wait_writes(b)

    plsc.subcore_barrier()
    pltpu.sync_copy(acc_sh.at[pl.ds(s * _RPT, _RPT)],
                    acc_out.at[c, pl.ds(s * _RPT, _RPT)])


def _sc2_body(val_out, dstb, acc_out,
              dstp_v, val_v, acc_sh, sva0, sva1, ssc0, ssc1):
    c = lax.axis_index("c")
    s = lax.axis_index("s")
    zero16 = jnp.zeros((16,), jnp.float32)
    sva = (sva0, sva1)
    ssc = (ssc0, ssc1)

    def _row0(g, jl):
        return s * _EPT + (g * _G2 + jl) * _B2

    def _zrow(r, carry):
        for i in range(_D // 16):
            val_v[0, r, pl.ds(i * 16, 16)] = zero16
        return carry
    lax.fori_loop(0, _B2, _zrow, 0)
    base = s * _ZPT
    for q in range(_ZPT // _B2):
        pltpu.sync_copy(val_v.at[0], acc_sh.at[pl.ds(base + q * _B2, _B2)])
    rem = _ZPT - (_ZPT // _B2) * _B2
    pltpu.sync_copy(val_v.at[0, pl.ds(0, rem)],
                    acc_sh.at[pl.ds(base + (_ZPT // _B2) * _B2, rem)])
    plsc.subcore_barrier()

    def _issue_val(g, jl, b):
        pltpu.async_copy(val_out.at[c, pl.ds(_row0(g, jl), _B2)], val_v.at[b],
                         sva[b])

    def _wait_val(b):
        pltpu.make_async_copy(val_out.at[c, pl.ds(0, _B2)], val_v.at[b],
                              sva[b]).wait()

    def _wait_sc(b):
        for q in range(2):
            pltpu.make_async_copy(val_v.at[b, pl.ds(0, _B2 // 2)],
                                  acc_sh.at[dstp_v.at[0]], ssc[b]).wait()

    pltpu.sync_copy(dstb.at[s, 0], dstp_v)
    _issue_val(0, 0, 0)

    def _group(g, carry):
        for jl in range(_G2):
            b = jl % 2

            @pl.when(jnp.logical_or(g > 0, jl >= 2))
            def _():
                _wait_sc(b)
            _wait_val(b)
            if jl + 1 < _G2:
                _issue_val(g, jl + 1, 1 - b)
            for q in range(2):
                pltpu.async_copy(val_v.at[b, pl.ds(q * (_B2 // 2), _B2 // 2)],
                                 acc_sh.at[dstp_v.at[jl * 2 + q]], ssc[b],
                                 add=True)

        @pl.when(g < _NG2 - 1)
        def _():
            pltpu.sync_copy(dstb.at[s, g + 1], dstp_v)
            _issue_val(g + 1, 0, 0)
        return carry
    lax.fori_loop(0, _NG2, _group, 0)
    for b in range(2):
        _wait_sc(b)

    plsc.subcore_barrier()
    pltpu.sync_copy(acc_sh.at[pl.ds(s * _RPT, _RPT)],
                    acc_out.at[c, pl.ds(s * _RPT, _RPT)])


def _sc_phase1(db2, eh2, ce2, srcg, dstf, dsta):
    mesh = plsc.VectorSubcoreMesh(core_axis_name="c", subcore_axis_name="s")
    fn = functools.partial(
        pl.kernel, _sc1_body, mesh=mesh,
        out_type=[
            jax.ShapeDtypeStruct((_NCORE, _E, _H), jnp.float32),
            jax.ShapeDtypeStruct((_NCORE, _E, _D), jnp.float32),
            jax.ShapeDtypeStruct((_NCORE, _PHN, _D), jnp.float32),
        ],
        scratch_types=[
            pltpu.VMEM((_G, _B), jnp.int32),
            pltpu.VMEM((_G, _B), jnp.int32),
            pltpu.VMEM((_G, _B), jnp.int32),
            pltpu.VMEM((2, _B, _D), jnp.float32),
            pltpu.VMEM((2, _B, _D), jnp.float32),
            pltpu.VMEM((2, _B, _H), jnp.float32),
            pltpu.VMEM((2, _B, _H), jnp.float32),
            pltpu.VMEM((2, _B, _D), jnp.float32),
            pltpu.VMEM_SHARED((_PACC, _D), jnp.float32),
        ] + [pltpu.SemaphoreType.DMA] * 12,
    )
    return fn()(db2, eh2, ce2, srcg, dstf, dsta)


def _sc_phase2(val, dstb):
    mesh = plsc.VectorSubcoreMesh(core_axis_name="c", subcore_axis_name="s")
    fn = functools.partial(
        pl.kernel, _sc2_body, mesh=mesh,
        out_type=jax.ShapeDtypeStruct((_NCORE, _PHN, _D), jnp.float32),
        scratch_types=[
            pltpu.VMEM((_G2 * 2, _B2 // 2), jnp.int32),
            pltpu.VMEM((2, _B2, _D), jnp.float32),
            pltpu.VMEM_SHARED((_PACC, _D), jnp.float32),
        ] + [pltpu.SemaphoreType.DMA] * 4,
    )
    return fn()(val, dstb)


def _estats_body(en_ref, out_ref, acc_ref):
    i = pl.program_id(0)

    @pl.when(i == 0)
    def _():
        acc_ref[...] = jnp.zeros_like(acc_ref)

    blk = jnp.concatenate([en_ref[0], en_ref[1]], axis=1)
    acc_ref[0:1] = acc_ref[0:1] + jnp.sum(blk, axis=0, keepdims=True)
    acc_ref[1:2] = acc_ref[1:2] + jnp.sum(blk * blk, axis=0, keepdims=True)

    @pl.when(i == _NBE - 1)
    def _():
        out_ref[...] = acc_ref[...]


def _estats(enew2):
    return pl.pallas_call(
        _estats_body,
        grid=(_NBE,),
        in_specs=[pl.BlockSpec((_NCORE, _BE, _H), lambda i: (0, i, 0))],
        out_specs=pl.BlockSpec((8, _D), lambda i: (0, 0)),
        out_shape=jax.ShapeDtypeStruct((8, _D), jnp.float32),
        scratch_shapes=[pltpu.VMEM((8, _D), jnp.float32)],
    )(enew2)


def _hpre_body(acc_in, ah_ref, hnp_ref, out_ref, st_ref):
    i = pl.program_id(0)

    @pl.when(i == 0)
    def _():
        st_ref[...] = jnp.zeros_like(st_ref)

    sh = jnp.concatenate([acc_in[0, :, :_H], acc_in[1, :, :_H]], axis=1)
    ss = jnp.concatenate([acc_in[0, :, _H:], acc_in[1, :, _H:]], axis=1)
    ah = jnp.concatenate([ah_ref[0], ah_ref[1]], axis=1)
    hnp = ah + sh / (ss + 1e-6)
    hnp_ref[...] = hnp
    st_ref[0:1] = st_ref[0:1] + jnp.sum(hnp, axis=0, keepdims=True)
    st_ref[1:2] = st_ref[1:2] + jnp.sum(hnp * hnp, axis=0, keepdims=True)

    @pl.when(i == _NBN - 1)
    def _():
        out_ref[...] = st_ref[...]


def _hpre(acc, Ah):
    return pl.pallas_call(
        _hpre_body,
        grid=(_NBN,),
        in_specs=[
            pl.BlockSpec((_NCORE, _BN_NODE, _D), lambda i: (0, i, 0)),
            pl.BlockSpec((_NCORE, _BN_NODE, _H), lambda i: (0, i, 0)),
        ],
        out_specs=[
            pl.BlockSpec((_BN_NODE, _D), lambda i: (i, 0)),
            pl.BlockSpec((8, _D), lambda i: (0, 0)),
        ],
        out_shape=[
            jax.ShapeDtypeStruct((_N, _D), jnp.float32),
            jax.ShapeDtypeStruct((8, _D), jnp.float32),
        ],
        scratch_shapes=[pltpu.VMEM((8, _D), jnp.float32)],
    )(acc, Ah)


def _bn_apply_body(nrows, x_ref, xn_ref, st_ref, g_ref, b_ref, out_ref):
    mu = st_ref[0:1] / nrows
    var = st_ref[1:2] / nrows - mu * mu
    xn = (xn_ref[...] - mu) * lax.rsqrt(var + 1e-5) * g_ref[...] + b_ref[...]
    out_ref[...] = x_ref[...] + jnp.maximum(xn, 0.0)


def _h_apply(h, hnp, hstats, g2, b2):
    return pl.pallas_call(
        functools.partial(_bn_apply_body, float(_N)),
        grid=(_NBN,),
        in_specs=[
            pl.BlockSpec((_BN_NODE, _D), lambda i: (i, 0)),
            pl.BlockSpec((_BN_NODE, _D), lambda i: (i, 0)),
            pl.BlockSpec((8, _D), lambda i: (0, 0)),
            pl.BlockSpec((1, _D), lambda i: (0, 0)),
            pl.BlockSpec((1, _D), lambda i: (0, 0)),
        ],
        out_specs=pl.BlockSpec((_BN_NODE, _D), lambda i: (i, 0)),
        out_shape=jax.ShapeDtypeStruct((_N, _D), jnp.float32),
    )(h, hnp, hstats, g2, b2)


def _e_apply_body(e_ref, en_ref, st_ref, g_ref, b_ref, out_ref):
    en = jnp.concatenate([en_ref[0], en_ref[1]], axis=1)
    mu = st_ref[0:1] / float(_E)
    var = st_ref[1:2] / float(_E) - mu * mu
    xn = (en - mu) * lax.rsqrt(var + 1e-5) * g_ref[...] + b_ref[...]
    out_ref[...] = e_ref[...] + jnp.maximum(xn, 0.0)


def _e_apply(e, enew2, estats, g2, b2):
    return pl.pallas_call(
        _e_apply_body,
        grid=(_NBE,),
        in_specs=[
            pl.BlockSpec((_BE, _D), lambda i: (i, 0)),
            pl.BlockSpec((_NCORE, _BE, _H), lambda i: (0, i, 0)),
            pl.BlockSpec((8, _D), lambda i: (0, 0)),
            pl.BlockSpec((1, _D), lambda i: (0, 0)),
            pl.BlockSpec((1, _D), lambda i: (0, 0)),
        ],
        out_specs=pl.BlockSpec((_BE, _D), lambda i: (i, 0)),
        out_shape=jax.ShapeDtypeStruct((_E, _D), jnp.float32),
    )(e, enew2, estats, g2, b2)


def kernel(h, e, edge_index, A_w, A_b, B_w, B_b, C_w, C_b, D_w, D_b, E_w, E_b,
           bn_h_gamma, bn_h_beta, bn_e_gamma, bn_e_beta):
    src = edge_index[0].reshape(_NSUB, _NG, _G, _B)
    dst = edge_index[1].reshape(_NSUB, _NG, _G, _B)
    srcg = jnp.stack([src, src + _N])
    dsta = jnp.where(dst < _PHN, dst, _PHN)
    dstb = jnp.where(dst >= _PHN, dst - _PHN, _PHN)
    dstb = dstb.reshape(_NSUB, _NG2, _G2 * 2, _B2 // 2)

    r1 = lambda v: v.reshape(1, _D)
    w2 = lambda W: W.reshape(_D, _NCORE, _H).transpose(1, 0, 2)
    b2 = lambda v: v.reshape(_NCORE, 1, _H)
    Ah, db2, eh2 = _node_mm(h, w2(A_w), b2(A_b), w2(B_w), b2(B_b),
                            w2(D_w), b2(D_b), E_w, r1(E_b))
    ce2 = _edge_mm(e, w2(C_w), b2(C_b))
    enew2, val, acca = _sc_phase1(db2, eh2, ce2, srcg, dst, dsta)
    accb = _sc_phase2(val, dstb)
    acc = jnp.concatenate([acca, accb], axis=1)
    estats = _estats(enew2)
    hnp, hstats = _hpre(acc, Ah)
    h_out = _h_apply(h, hnp, hstats, r1(bn_h_gamma), r1(bn_h_beta))
    e_out = _e_apply(e, enew2, estats, r1(bn_e_gamma), r1(bn_e_beta))
    return (h_out, e_out)
